# baseline (speedup 1.0000x reference)
import functools

import jax
import jax.numpy as jnp
from jax import lax
from jax.experimental import pallas as pl
from jax.experimental.pallas import tpu as pltpu
from jax.experimental.pallas import tpu_sc as plsc

_N = 10000
_E = 320000
_DIM = 128
_NET = 4
_NSTEPS = 4

_NC = 2
_NS = 16
_NW = _NC * _NS
_EPW = _E // _NW
_C = 80
_NCH = _EPW // _C
_RPT = 624
_REM = _N - _NS * _RPT


def _msg_body(h_ref, w_ref, b_ref, o_ref):
    o_ref[...] = (
        jnp.dot(h_ref[...], w_ref[...], preferred_element_type=jnp.float32)
        + b_ref[...]
    )


def _msg(h, w_t, b2d):
    rb = 1000
    return pl.pallas_call(
        _msg_body,
        grid=(_N // rb,),
        in_specs=[
            pl.BlockSpec((rb, _DIM), lambda i: (i, 0)),
            pl.BlockSpec((_DIM, _NET * _DIM), lambda i: (0, 0)),
            pl.BlockSpec((1, _NET * _DIM), lambda i: (0, 0)),
        ],
        out_specs=pl.BlockSpec((rb, _NET * _DIM), lambda i: (i, 0)),
        out_shape=jax.ShapeDtypeStruct((_N, _NET * _DIM), jnp.float32),
    )(h, w_t, b2d)


@functools.partial(
    pl.kernel,
    out_type=jax.ShapeDtypeStruct((_NC, _N, _DIM), jnp.float32),
    mesh=plsc.VectorSubcoreMesh(core_axis_name="c", subcore_axis_name="s"),
    scratch_types=[
        pltpu.VMEM((_NCH, _C), jnp.int32),
        pltpu.VMEM((2, _C), jnp.int32),
        pltpu.VMEM((2, _C), jnp.int32),
        pltpu.VMEM((2, _C), jnp.int32),
        pltpu.VMEM((_C, _DIM), jnp.float32),
        pltpu.VMEM((_C, _DIM), jnp.float32),
        pltpu.VMEM((_C, _DIM), jnp.float32),
        pltpu.VMEM_SHARED((_N, _DIM), jnp.float32),
    ] + [pltpu.SemaphoreType.DMA] * 6,
)
def _sc_agg(m_hbm, idx_hbm, z_hbm, out_hbm,
            idxv, sdb0, sdb1, sdb2, rb0, rb1, rb2, aggsh,
            gs0, gs1, gs2, ss0, ss1, ss2):
    cid = lax.axis_index("c")
    sid = lax.axis_index("s")
    wid = sid * _NC + cid

    pltpu.sync_copy(idx_hbm.at[wid], idxv)

    def _unpack(jj, sdb):
        for k in range(_C // 16):
            x = idxv[jj, pl.ds(k * 16, 16)]
            sdb[0, pl.ds(k * 16, 16)] = lax.bitwise_and(x, 0xFFFF)
            sdb[1, pl.ds(k * 16, 16)] = lax.shift_right_logical(x, 16)

    pltpu.sync_copy(z_hbm, aggsh.at[pl.ds(sid * _RPT, _RPT)])

    @pl.when(sid == 0)
    def _():
        pltpu.sync_copy(
            z_hbm.at[pl.ds(0, _REM)], aggsh.at[pl.ds(_NS * _RPT, _REM)]
        )

    slots = (
        (sdb0, rb0, gs0, ss0),
        (sdb1, rb1, gs1, ss1),
        (sdb2, rb2, gs2, ss2),
    )

    _unpack(0, sdb0)
    pltpu.async_copy(m_hbm.at[sdb0.at[0]], rb0, gs0)

    plsc.subcore_barrier()

    def _body(k, r):
        sdb, rb, gs, ss = slots[r]
        sdb1, rb1, gs1, ss1_ = slots[(r + 1) % 3]
        pltpu.make_async_copy(m_hbm.at[sdb.at[0]], rb, gs).wait()
        pltpu.async_copy(rb, aggsh.at[sdb.at[1]], ss, add=True)

        @pl.when(k >= 2)
        def _():
            pltpu.make_async_copy(rb1, aggsh.at[sdb1.at[1]], ss1_).wait()

        @pl.when(k + 1 < _NCH)
        def _():
            _unpack(k + 1, sdb1)
            pltpu.async_copy(m_hbm.at[sdb1.at[0]], rb1, gs1)

    def _tri(i3, carry):
        k0 = i3 * 3
        for r in range(3):
            _body(k0 + r, r)
        return carry

    _ntri = (_NCH - 2) // 3
    assert _ntri * 3 + 2 == _NCH
    lax.fori_loop(0, _ntri, _tri, 0)
    _body(_NCH - 2, 0)
    _body(_NCH - 1, 1)
    pltpu.make_async_copy(rb0, aggsh.at[sdb0.at[1]], ss0).wait()
    pltpu.make_async_copy(rb1, aggsh.at[sdb1.at[1]], ss1).wait()

    plsc.subcore_barrier()
    pltpu.sync_copy(
        aggsh.at[pl.ds(sid * _RPT, _RPT)],
        out_hbm.at[cid, pl.ds(sid * _RPT, _RPT)],
    )

    @pl.when(sid == 0)
    def _():
        pltpu.sync_copy(
            aggsh.at[pl.ds(_NS * _RPT, _REM)],
            out_hbm.at[cid, pl.ds(_NS * _RPT, _REM)],
        )


def _gru_msg_body(a_ref, h_ref, wih_ref, whh_ref, bih_ref, bhh_ref,
                  wm_ref, bm_ref, o_ref, m_ref):
    agg = a_ref[0] + a_ref[1]
    h = h_ref[...]
    gi = (
        jnp.dot(agg, wih_ref[...], preferred_element_type=jnp.float32)
        + bih_ref[...]
    )
    gh = (
        jnp.dot(h, whh_ref[...], preferred_element_type=jnp.float32)
        + bhh_ref[...]
    )
    rg = jax.nn.sigmoid(gi[:, :_DIM] + gh[:, :_DIM])
    zg = jax.nn.sigmoid(gi[:, _DIM:2 * _DIM] + gh[:, _DIM:2 * _DIM])
    ng = jnp.tanh(gi[:, 2 * _DIM:] + rg * gh[:, 2 * _DIM:])
    h_new = (1.0 - zg) * ng + zg * h
    o_ref[...] = h_new
    m_ref[...] = (
        jnp.dot(h_new, wm_ref[...], preferred_element_type=jnp.float32)
        + bm_ref[...]
    )


def _gru_msg(agg2, h, wih_t, whh_t, bih2d, bhh2d, wm_t, bm2d):
    rb = 1000
    return pl.pallas_call(
        _gru_msg_body,
        grid=(_N // rb,),
        in_specs=[
            pl.BlockSpec((_NC, rb, _DIM), lambda i: (0, i, 0)),
            pl.BlockSpec((rb, _DIM), lambda i: (i, 0)),
            pl.BlockSpec((_DIM, 3 * _DIM), lambda i: (0, 0)),
            pl.BlockSpec((_DIM, 3 * _DIM), lambda i: (0, 0)),
            pl.BlockSpec((1, 3 * _DIM), lambda i: (0, 0)),
            pl.BlockSpec((1, 3 * _DIM), lambda i: (0, 0)),
            pl.BlockSpec((_DIM, _NET * _DIM), lambda i: (0, 0)),
            pl.BlockSpec((1, _NET * _DIM), lambda i: (0, 0)),
        ],
        out_specs=[
            pl.BlockSpec((rb, _DIM), lambda i: (i, 0)),
            pl.BlockSpec((rb, _NET * _DIM), lambda i: (i, 0)),
        ],
        out_shape=[
            jax.ShapeDtypeStruct((_N, _DIM), jnp.float32),
            jax.ShapeDtypeStruct((_N, _NET * _DIM), jnp.float32),
        ],
    )(agg2, h, wih_t, whh_t, bih2d, bhh2d, wm_t, bm2d)


def _gru_body(a_ref, h_ref, wih_ref, whh_ref, bih_ref, bhh_ref, o_ref):
    agg = a_ref[0] + a_ref[1]
    h = h_ref[...]
    gi = (
        jnp.dot(agg, wih_ref[...], preferred_element_type=jnp.float32)
        + bih_ref[...]
    )
    gh = (
        jnp.dot(h, whh_ref[...], preferred_element_type=jnp.float32)
        + bhh_ref[...]
    )
    rg = jax.nn.sigmoid(gi[:, :_DIM] + gh[:, :_DIM])
    zg = jax.nn.sigmoid(gi[:, _DIM:2 * _DIM] + gh[:, _DIM:2 * _DIM])
    ng = jnp.tanh(gi[:, 2 * _DIM:] + rg * gh[:, 2 * _DIM:])
    o_ref[...] = (1.0 - zg) * ng + zg * h


def _gru(agg2, h, wih_t, whh_t, bih2d, bhh2d):
    rb = 1000
    return pl.pallas_call(
        _gru_body,
        grid=(_N // rb,),
        in_specs=[
            pl.BlockSpec((_NC, rb, _DIM), lambda i: (0, i, 0)),
            pl.BlockSpec((rb, _DIM), lambda i: (i, 0)),
            pl.BlockSpec((_DIM, 3 * _DIM), lambda i: (0, 0)),
            pl.BlockSpec((_DIM, 3 * _DIM), lambda i: (0, 0)),
            pl.BlockSpec((1, 3 * _DIM), lambda i: (0, 0)),
            pl.BlockSpec((1, 3 * _DIM), lambda i: (0, 0)),
        ],
        out_specs=pl.BlockSpec((rb, _DIM), lambda i: (i, 0)),
        out_shape=jax.ShapeDtypeStruct((_N, _DIM), jnp.float32),
    )(agg2, h, wih_t, whh_t, bih2d, bhh2d)


def kernel(embedding, edges, W_msg, b_msg, W_ih, W_hh, b_ih, b_hh):
    wm_t = W_msg.T
    wih_t = W_ih.T
    whh_t = W_hh.T
    bm2d = b_msg.reshape(1, _NET * _DIM)
    bih2d = b_ih.reshape(1, 3 * _DIM)
    bhh2d = b_hh.reshape(1, 3 * _DIM)

    e = edges.astype(jnp.int32)
    src = e[:, 0] * _NET + e[:, 2]
    dst = e[:, 1]
    packed = jnp.bitwise_or(jnp.left_shift(dst, 16), src).reshape(
        _NW, _NCH, _C)
    zrows = jnp.zeros((_RPT, _DIM), jnp.float32)

    h = embedding
    m = _msg(h, wm_t, bm2d)
    for step in range(_NSTEPS):
        agg2 = _sc_agg(m.reshape(_N * _NET, _DIM), packed, zrows)
        if step < _NSTEPS - 1:
            h, m = _gru_msg(agg2, h, wih_t, whh_t, bih2d, bhh2d, wm_t, bm2d)
        else:
            h = _gru(agg2, h, wih_t, whh_t, bih2d, bhh2d)
    return h


# speedup vs baseline: 1.3946x; 1.3946x over previous
import functools

import jax
import jax.numpy as jnp
from jax import lax
from jax.experimental import pallas as pl
from jax.experimental.pallas import tpu as pltpu
from jax.experimental.pallas import tpu_sc as plsc

_N = 10000
_E = 320000
_DIM = 128
_NET = 4
_NSTEPS = 4

_NC = 2
_NS = 16
_NW = _NC * _NS
_EPW = _E // _NW
_C = 80
_NCH = _EPW // _C
_RPT = 624
_REM = _N - _NS * _RPT


def _msg_body(h_ref, w_ref, b_ref, o_ref):
    o_ref[...] = (
        jnp.dot(h_ref[...], w_ref[...], preferred_element_type=jnp.float32)
        + b_ref[...]
    )


def _msg(h, w_t, b2d):
    rb = 1000
    return pl.pallas_call(
        _msg_body,
        grid=(_N // rb,),
        in_specs=[
            pl.BlockSpec((rb, _DIM), lambda i: (i, 0)),
            pl.BlockSpec((_DIM, _NET * _DIM), lambda i: (0, 0)),
            pl.BlockSpec((1, _NET * _DIM), lambda i: (0, 0)),
        ],
        out_specs=pl.BlockSpec((rb, _NET * _DIM), lambda i: (i, 0)),
        out_shape=jax.ShapeDtypeStruct((_N, _NET * _DIM), jnp.float32),
    )(h, w_t, b2d)


@functools.partial(
    pl.kernel,
    out_type=jax.ShapeDtypeStruct((_NC, _N, _DIM), jnp.float32),
    mesh=plsc.VectorSubcoreMesh(core_axis_name="c", subcore_axis_name="s"),
    scratch_types=[
        pltpu.VMEM((_NCH, _C), jnp.int32),
        pltpu.VMEM((2, _C), jnp.int32),
        pltpu.VMEM((2, _C), jnp.int32),
        pltpu.VMEM((2, _C), jnp.int32),
        pltpu.VMEM((_C, _DIM), jnp.float32),
        pltpu.VMEM((_C, _DIM), jnp.float32),
        pltpu.VMEM((_C, _DIM), jnp.float32),
        pltpu.VMEM_SHARED((_N, _DIM), jnp.float32),
    ] + [pltpu.SemaphoreType.DMA] * 7,
)
def _sc_agg(m_hbm, idx_hbm, z_hbm, out_hbm,
            idxv, sdb0, sdb1, sdb2, rb0, rb1, rb2, aggsh,
            gs0, gs1, gs2, ss0, ss1, ss2, zsem):
    cid = lax.axis_index("c")
    sid = lax.axis_index("s")
    wid = sid * _NC + cid

    pltpu.async_copy(z_hbm, aggsh.at[pl.ds(sid * _RPT, _RPT)], zsem)

    pltpu.sync_copy(idx_hbm.at[wid], idxv)

    def _unpack(jj, sdb):
        for k in range(_C // 16):
            x = idxv[jj, pl.ds(k * 16, 16)]
            sdb[0, pl.ds(k * 16, 16)] = lax.bitwise_and(x, 0xFFFF)
            sdb[1, pl.ds(k * 16, 16)] = lax.shift_right_logical(x, 16)

    @pl.when(sid == 0)
    def _():
        pltpu.sync_copy(
            z_hbm.at[pl.ds(0, _REM)], aggsh.at[pl.ds(_NS * _RPT, _REM)]
        )

    slots = (
        (sdb0, rb0, gs0, ss0),
        (sdb1, rb1, gs1, ss1),
        (sdb2, rb2, gs2, ss2),
    )

    _unpack(0, sdb0)
    pltpu.async_copy(m_hbm.at[sdb0.at[0]], rb0, gs0)
    _unpack(1, sdb1)
    pltpu.async_copy(m_hbm.at[sdb1.at[0]], rb1, gs1)

    pltpu.make_async_copy(z_hbm, aggsh.at[pl.ds(sid * _RPT, _RPT)], zsem).wait()
    plsc.subcore_barrier()

    def _body(k, r):
        sdb, rb, gs, ss = slots[r]
        sdb2, rb2, gs2, ss2_ = slots[(r + 2) % 3]
        pltpu.make_async_copy(m_hbm.at[sdb.at[0]], rb, gs).wait()
        pltpu.async_copy(rb, aggsh.at[sdb.at[1]], ss, add=True)

        @pl.when(k >= 1)
        def _():
            pltpu.make_async_copy(rb2, aggsh.at[sdb2.at[1]], ss2_).wait()

        @pl.when(k + 2 < _NCH)
        def _():
            _unpack(k + 2, sdb2)
            pltpu.async_copy(m_hbm.at[sdb2.at[0]], rb2, gs2)

    def _tri(i3, carry):
        k0 = i3 * 3
        for r in range(3):
            _body(k0 + r, r)
        return carry

    _ntri = (_NCH - 2) // 3
    assert _ntri * 3 + 2 == _NCH
    lax.fori_loop(0, _ntri, _tri, 0)
    _body(_NCH - 2, 0)
    _body(_NCH - 1, 1)
    pltpu.make_async_copy(rb1, aggsh.at[sdb1.at[1]], ss1).wait()

    plsc.subcore_barrier()
    pltpu.sync_copy(
        aggsh.at[pl.ds(sid * _RPT, _RPT)],
        out_hbm.at[cid, pl.ds(sid * _RPT, _RPT)],
    )

    @pl.when(sid == 0)
    def _():
        pltpu.sync_copy(
            aggsh.at[pl.ds(_NS * _RPT, _REM)],
            out_hbm.at[cid, pl.ds(_NS * _RPT, _REM)],
        )


def _gru_msg_body(a_ref, h_ref, wih_ref, whh_ref, bih_ref, bhh_ref,
                  wm_ref, bm_ref, o_ref, m_ref):
    agg = a_ref[0] + a_ref[1]
    h = h_ref[...]
    gi = (
        jnp.dot(agg, wih_ref[...], preferred_element_type=jnp.float32)
        + bih_ref[...]
    )
    gh = (
        jnp.dot(h, whh_ref[...], preferred_element_type=jnp.float32)
        + bhh_ref[...]
    )
    rg = jax.nn.sigmoid(gi[:, :_DIM] + gh[:, :_DIM])
    zg = jax.nn.sigmoid(gi[:, _DIM:2 * _DIM] + gh[:, _DIM:2 * _DIM])
    ng = jnp.tanh(gi[:, 2 * _DIM:] + rg * gh[:, 2 * _DIM:])
    h_new = (1.0 - zg) * ng + zg * h
    o_ref[...] = h_new
    m_ref[...] = (
        jnp.dot(h_new, wm_ref[...], preferred_element_type=jnp.float32)
        + bm_ref[...]
    )


def _gru_msg(agg2, h, wih_t, whh_t, bih2d, bhh2d, wm_t, bm2d):
    rb = 1000
    return pl.pallas_call(
        _gru_msg_body,
        grid=(_N // rb,),
        in_specs=[
            pl.BlockSpec((_NC, rb, _DIM), lambda i: (0, i, 0)),
            pl.BlockSpec((rb, _DIM), lambda i: (i, 0)),
            pl.BlockSpec((_DIM, 3 * _DIM), lambda i: (0, 0)),
            pl.BlockSpec((_DIM, 3 * _DIM), lambda i: (0, 0)),
            pl.BlockSpec((1, 3 * _DIM), lambda i: (0, 0)),
            pl.BlockSpec((1, 3 * _DIM), lambda i: (0, 0)),
            pl.BlockSpec((_DIM, _NET * _DIM), lambda i: (0, 0)),
            pl.BlockSpec((1, _NET * _DIM), lambda i: (0, 0)),
        ],
        out_specs=[
            pl.BlockSpec((rb, _DIM), lambda i: (i, 0)),
            pl.BlockSpec((rb, _NET * _DIM), lambda i: (i, 0)),
        ],
        out_shape=[
            jax.ShapeDtypeStruct((_N, _DIM), jnp.float32),
            jax.ShapeDtypeStruct((_N, _NET * _DIM), jnp.float32),
        ],
    )(agg2, h, wih_t, whh_t, bih2d, bhh2d, wm_t, bm2d)


def _gru_body(a_ref, h_ref, wih_ref, whh_ref, bih_ref, bhh_ref, o_ref):
    agg = a_ref[0] + a_ref[1]
    h = h_ref[...]
    gi = (
        jnp.dot(agg, wih_ref[...], preferred_element_type=jnp.float32)
        + bih_ref[...]
    )
    gh = (
        jnp.dot(h, whh_ref[...], preferred_element_type=jnp.float32)
        + bhh_ref[...]
    )
    rg = jax.nn.sigmoid(gi[:, :_DIM] + gh[:, :_DIM])
    zg = jax.nn.sigmoid(gi[:, _DIM:2 * _DIM] + gh[:, _DIM:2 * _DIM])
    ng = jnp.tanh(gi[:, 2 * _DIM:] + rg * gh[:, 2 * _DIM:])
    o_ref[...] = (1.0 - zg) * ng + zg * h


def _gru(agg2, h, wih_t, whh_t, bih2d, bhh2d):
    rb = 1000
    return pl.pallas_call(
        _gru_body,
        grid=(_N // rb,),
        in_specs=[
            pl.BlockSpec((_NC, rb, _DIM), lambda i: (0, i, 0)),
            pl.BlockSpec((rb, _DIM), lambda i: (i, 0)),
            pl.BlockSpec((_DIM, 3 * _DIM), lambda i: (0, 0)),
            pl.BlockSpec((_DIM, 3 * _DIM), lambda i: (0, 0)),
            pl.BlockSpec((1, 3 * _DIM), lambda i: (0, 0)),
            pl.BlockSpec((1, 3 * _DIM), lambda i: (0, 0)),
        ],
        out_specs=pl.BlockSpec((rb, _DIM), lambda i: (i, 0)),
        out_shape=jax.ShapeDtypeStruct((_N, _DIM), jnp.float32),
    )(agg2, h, wih_t, whh_t, bih2d, bhh2d)


def kernel(embedding, edges, W_msg, b_msg, W_ih, W_hh, b_ih, b_hh):
    wm_t = W_msg.T
    wih_t = W_ih.T
    whh_t = W_hh.T
    bm2d = b_msg.reshape(1, _NET * _DIM)
    bih2d = b_ih.reshape(1, 3 * _DIM)
    bhh2d = b_hh.reshape(1, 3 * _DIM)

    e = edges.astype(jnp.int32)
    src = e[:, 0] * _NET + e[:, 2]
    dst = e[:, 1]
    packed = jnp.bitwise_or(jnp.left_shift(dst, 16), src).reshape(
        _NW, _NCH, _C)
    zrows = jnp.zeros((_RPT, _DIM), jnp.float32)

    h = embedding
    m = _msg(h, wm_t, bm2d)
    for step in range(_NSTEPS):
        agg2 = _sc_agg(m.reshape(_N * _NET, _DIM), packed, zrows)
        if step < _NSTEPS - 1:
            h, m = _gru_msg(agg2, h, wih_t, whh_t, bih2d, bhh2d, wm_t, bm2d)
        else:
            h = _gru(agg2, h, wih_t, whh_t, bih2d, bhh2d)
    return h


# speedup vs baseline: 1.3980x; 1.0024x over previous
import functools

import jax
import jax.numpy as jnp
from jax import lax
from jax.experimental import pallas as pl
from jax.experimental.pallas import tpu as pltpu
from jax.experimental.pallas import tpu_sc as plsc

_N = 10000
_E = 320000
_DIM = 128
_NET = 4
_NSTEPS = 4

_NC = 2
_NS = 16
_NW = _NC * _NS
_EPW = _E // _NW
_C = 80
_NCH = _EPW // _C
_RPT = 624
_REM = _N - _NS * _RPT


def _msg_body(h_ref, w_ref, b_ref, o_ref):
    o_ref[...] = (
        jnp.dot(h_ref[...], w_ref[...], preferred_element_type=jnp.float32)
        + b_ref[...]
    )


def _msg(h, w_t, b2d):
    rb = 1000
    return pl.pallas_call(
        _msg_body,
        grid=(_N // rb,),
        in_specs=[
            pl.BlockSpec((rb, _DIM), lambda i: (i, 0)),
            pl.BlockSpec((_DIM, _NET * _DIM), lambda i: (0, 0)),
            pl.BlockSpec((1, _NET * _DIM), lambda i: (0, 0)),
        ],
        out_specs=pl.BlockSpec((rb, _NET * _DIM), lambda i: (i, 0)),
        out_shape=jax.ShapeDtypeStruct((_N, _NET * _DIM), jnp.float32),
    )(h, w_t, b2d)


@functools.partial(
    pl.kernel,
    out_type=jax.ShapeDtypeStruct((_NC, _N, _DIM), jnp.float32),
    mesh=plsc.VectorSubcoreMesh(core_axis_name="c", subcore_axis_name="s"),
    scratch_types=[
        pltpu.VMEM((_NCH, _C), jnp.int32),
        pltpu.VMEM((2, _C), jnp.int32),
        pltpu.VMEM((2, _C), jnp.int32),
        pltpu.VMEM((2, _C), jnp.int32),
        pltpu.VMEM((_C, _DIM), jnp.float32),
        pltpu.VMEM((_C, _DIM), jnp.float32),
        pltpu.VMEM((_C, _DIM), jnp.float32),
        pltpu.VMEM_SHARED((_N, _DIM), jnp.float32),
    ] + [pltpu.SemaphoreType.DMA] * 10,
)
def _sc_agg(m_hbm, idx_hbm, z_hbm, out_hbm,
            idxv, sdb0, sdb1, sdb2, rb0, rb1, rb2, aggsh,
            gs0, gs1, gs2, gt0, gt1, gt2, ss0, ss1, ss2, zsem):
    cid = lax.axis_index("c")
    sid = lax.axis_index("s")
    wid = sid * _NC + cid

    pltpu.async_copy(z_hbm, aggsh.at[pl.ds(sid * _RPT, _RPT)], zsem)

    pltpu.sync_copy(idx_hbm.at[wid], idxv)

    def _unpack(jj, sdb):
        for k in range(_C // 16):
            x = idxv[jj, pl.ds(k * 16, 16)]
            sdb[0, pl.ds(k * 16, 16)] = lax.bitwise_and(x, 0xFFFF)
            sdb[1, pl.ds(k * 16, 16)] = lax.shift_right_logical(x, 16)

    @pl.when(sid == 0)
    def _():
        pltpu.sync_copy(
            z_hbm.at[pl.ds(0, _REM)], aggsh.at[pl.ds(_NS * _RPT, _REM)]
        )

    slots = (
        (sdb0, rb0, gs0, gt0, ss0),
        (sdb1, rb1, gs1, gt1, ss1),
        (sdb2, rb2, gs2, gt2, ss2),
    )
    _H = _C // 2

    def _fire_gather(sdb, rb, gs, gt):
        pltpu.async_copy(
            m_hbm.at[sdb.at[0, pl.ds(0, _H)]], rb.at[pl.ds(0, _H)], gs)
        pltpu.async_copy(
            m_hbm.at[sdb.at[0, pl.ds(_H, _H)]], rb.at[pl.ds(_H, _H)], gt)

    def _wait_gather(sdb, rb, gs, gt):
        pltpu.make_async_copy(
            m_hbm.at[sdb.at[0, pl.ds(0, _H)]], rb.at[pl.ds(0, _H)], gs).wait()
        pltpu.make_async_copy(
            m_hbm.at[sdb.at[0, pl.ds(_H, _H)]], rb.at[pl.ds(_H, _H)], gt).wait()

    _unpack(0, sdb0)
    _fire_gather(sdb0, rb0, gs0, gt0)
    _unpack(1, sdb1)
    _fire_gather(sdb1, rb1, gs1, gt1)

    pltpu.make_async_copy(z_hbm, aggsh.at[pl.ds(sid * _RPT, _RPT)], zsem).wait()
    plsc.subcore_barrier()

    def _body(k, r):
        sdb, rb, gs, gt, ss = slots[r]
        sdb2, rb2, gs2, gt2_, ss2_ = slots[(r + 2) % 3]
        _wait_gather(sdb, rb, gs, gt)
        pltpu.async_copy(rb, aggsh.at[sdb.at[1]], ss, add=True)

        @pl.when(k >= 1)
        def _():
            pltpu.make_async_copy(rb2, aggsh.at[sdb2.at[1]], ss2_).wait()

        @pl.when(k + 2 < _NCH)
        def _():
            _unpack(k + 2, sdb2)
            _fire_gather(sdb2, rb2, gs2, gt2_)

    def _tri(i3, carry):
        k0 = i3 * 3
        for r in range(3):
            _body(k0 + r, r)
        return carry

    _ntri = (_NCH - 2) // 3
    assert _ntri * 3 + 2 == _NCH
    lax.fori_loop(0, _ntri, _tri, 0)
    _body(_NCH - 2, 0)
    _body(_NCH - 1, 1)
    pltpu.make_async_copy(rb1, aggsh.at[sdb1.at[1]], ss1).wait()

    plsc.subcore_barrier()
    pltpu.sync_copy(
        aggsh.at[pl.ds(sid * _RPT, _RPT)],
        out_hbm.at[cid, pl.ds(sid * _RPT, _RPT)],
    )

    @pl.when(sid == 0)
    def _():
        pltpu.sync_copy(
            aggsh.at[pl.ds(_NS * _RPT, _REM)],
            out_hbm.at[cid, pl.ds(_NS * _RPT, _REM)],
        )


def _gru_msg_body(a_ref, h_ref, wih_ref, whh_ref, bih_ref, bhh_ref,
                  wm_ref, bm_ref, o_ref, m_ref):
    agg = a_ref[0] + a_ref[1]
    h = h_ref[...]
    gi = (
        jnp.dot(agg, wih_ref[...], preferred_element_type=jnp.float32)
        + bih_ref[...]
    )
    gh = (
        jnp.dot(h, whh_ref[...], preferred_element_type=jnp.float32)
        + bhh_ref[...]
    )
    rg = jax.nn.sigmoid(gi[:, :_DIM] + gh[:, :_DIM])
    zg = jax.nn.sigmoid(gi[:, _DIM:2 * _DIM] + gh[:, _DIM:2 * _DIM])
    ng = jnp.tanh(gi[:, 2 * _DIM:] + rg * gh[:, 2 * _DIM:])
    h_new = (1.0 - zg) * ng + zg * h
    o_ref[...] = h_new
    m_ref[...] = (
        jnp.dot(h_new, wm_ref[...], preferred_element_type=jnp.float32)
        + bm_ref[...]
    )


def _gru_msg(agg2, h, wih_t, whh_t, bih2d, bhh2d, wm_t, bm2d):
    rb = 1000
    return pl.pallas_call(
        _gru_msg_body,
        grid=(_N // rb,),
        in_specs=[
            pl.BlockSpec((_NC, rb, _DIM), lambda i: (0, i, 0)),
            pl.BlockSpec((rb, _DIM), lambda i: (i, 0)),
            pl.BlockSpec((_DIM, 3 * _DIM), lambda i: (0, 0)),
            pl.BlockSpec((_DIM, 3 * _DIM), lambda i: (0, 0)),
            pl.BlockSpec((1, 3 * _DIM), lambda i: (0, 0)),
            pl.BlockSpec((1, 3 * _DIM), lambda i: (0, 0)),
            pl.BlockSpec((_DIM, _NET * _DIM), lambda i: (0, 0)),
            pl.BlockSpec((1, _NET * _DIM), lambda i: (0, 0)),
        ],
        out_specs=[
            pl.BlockSpec((rb, _DIM), lambda i: (i, 0)),
            pl.BlockSpec((rb, _NET * _DIM), lambda i: (i, 0)),
        ],
        out_shape=[
            jax.ShapeDtypeStruct((_N, _DIM), jnp.float32),
            jax.ShapeDtypeStruct((_N, _NET * _DIM), jnp.float32),
        ],
    )(agg2, h, wih_t, whh_t, bih2d, bhh2d, wm_t, bm2d)


def _gru_body(a_ref, h_ref, wih_ref, whh_ref, bih_ref, bhh_ref, o_ref):
    agg = a_ref[0] + a_ref[1]
    h = h_ref[...]
    gi = (
        jnp.dot(agg, wih_ref[...], preferred_element_type=jnp.float32)
        + bih_ref[...]
    )
    gh = (
        jnp.dot(h, whh_ref[...], preferred_element_type=jnp.float32)
        + bhh_ref[...]
    )
    rg = jax.nn.sigmoid(gi[:, :_DIM] + gh[:, :_DIM])
    zg = jax.nn.sigmoid(gi[:, _DIM:2 * _DIM] + gh[:, _DIM:2 * _DIM])
    ng = jnp.tanh(gi[:, 2 * _DIM:] + rg * gh[:, 2 * _DIM:])
    o_ref[...] = (1.0 - zg) * ng + zg * h


def _gru(agg2, h, wih_t, whh_t, bih2d, bhh2d):
    rb = 1000
    return pl.pallas_call(
        _gru_body,
        grid=(_N // rb,),
        in_specs=[
            pl.BlockSpec((_NC, rb, _DIM), lambda i: (0, i, 0)),
            pl.BlockSpec((rb, _DIM), lambda i: (i, 0)),
            pl.BlockSpec((_DIM, 3 * _DIM), lambda i: (0, 0)),
            pl.BlockSpec((_DIM, 3 * _DIM), lambda i: (0, 0)),
            pl.BlockSpec((1, 3 * _DIM), lambda i: (0, 0)),
            pl.BlockSpec((1, 3 * _DIM), lambda i: (0, 0)),
        ],
        out_specs=pl.BlockSpec((rb, _DIM), lambda i: (i, 0)),
        out_shape=jax.ShapeDtypeStruct((_N, _DIM), jnp.float32),
    )(agg2, h, wih_t, whh_t, bih2d, bhh2d)


def kernel(embedding, edges, W_msg, b_msg, W_ih, W_hh, b_ih, b_hh):
    wm_t = W_msg.T
    wih_t = W_ih.T
    whh_t = W_hh.T
    bm2d = b_msg.reshape(1, _NET * _DIM)
    bih2d = b_ih.reshape(1, 3 * _DIM)
    bhh2d = b_hh.reshape(1, 3 * _DIM)

    e = edges.astype(jnp.int32)
    src = e[:, 0] * _NET + e[:, 2]
    dst = e[:, 1]
    packed = jnp.bitwise_or(jnp.left_shift(dst, 16), src).reshape(
        _NW, _NCH, _C)
    zrows = jnp.zeros((_RPT, _DIM), jnp.float32)

    h = embedding
    m = _msg(h, wm_t, bm2d)
    for step in range(_NSTEPS):
        agg2 = _sc_agg(m.reshape(_N * _NET, _DIM), packed, zrows)
        if step < _NSTEPS - 1:
            h, m = _gru_msg(agg2, h, wih_t, whh_t, bih2d, bhh2d, wm_t, bm2d)
        else:
            h = _gru(agg2, h, wih_t, whh_t, bih2d, bhh2d)
    return h


# speedup vs baseline: 1.6185x; 1.1578x over previous
import functools

import jax
import jax.numpy as jnp
from jax import lax
from jax.experimental import pallas as pl
from jax.experimental.pallas import tpu as pltpu
from jax.experimental.pallas import tpu_sc as plsc

_N = 10000
_E = 320000
_DIM = 128
_NET = 4
_NSTEPS = 4

_NC = 2
_NS = 16
_NW = _NC * _NS
_EPW = _E // _NW
_C = 80
_NCH = _EPW // _C
_RPT = 624
_REM = _N - _NS * _RPT


def _msg_body(h_ref, w_ref, b_ref, o_ref):
    h = h_ref[...]
    for e in range(_NET):
        o_ref[e] = (
            jnp.dot(h, w_ref[:, e * _DIM:(e + 1) * _DIM],
                    preferred_element_type=jnp.float32)
            + b_ref[:, e * _DIM:(e + 1) * _DIM]
        )


def _msg(h, w_t, b2d):
    rb = 1000
    return pl.pallas_call(
        _msg_body,
        grid=(_N // rb,),
        in_specs=[
            pl.BlockSpec((rb, _DIM), lambda i: (i, 0)),
            pl.BlockSpec((_DIM, _NET * _DIM), lambda i: (0, 0)),
            pl.BlockSpec((1, _NET * _DIM), lambda i: (0, 0)),
        ],
        out_specs=pl.BlockSpec((_NET, rb, _DIM), lambda i: (0, i, 0)),
        out_shape=jax.ShapeDtypeStruct((_NET, _N, _DIM), jnp.float32),
    )(h, w_t, b2d)


@functools.partial(
    pl.kernel,
    out_type=jax.ShapeDtypeStruct((_NC, _N, _DIM), jnp.float32),
    mesh=plsc.VectorSubcoreMesh(core_axis_name="c", subcore_axis_name="s"),
    scratch_types=[
        pltpu.VMEM((_NCH, _C), jnp.int32),
        pltpu.VMEM((2, _C), jnp.int32),
        pltpu.VMEM((2, _C), jnp.int32),
        pltpu.VMEM((2, _C), jnp.int32),
        pltpu.VMEM((_C, _DIM), jnp.float32),
        pltpu.VMEM((_C, _DIM), jnp.float32),
        pltpu.VMEM((_C, _DIM), jnp.float32),
        pltpu.VMEM_SHARED((_N, _DIM), jnp.float32),
    ] + [pltpu.SemaphoreType.DMA] * 7,
)
def _sc_agg(m_hbm, idx_hbm, z_hbm, out_hbm,
            idxv, sdb0, sdb1, sdb2, rb0, rb1, rb2, aggsh,
            gs0, gs1, gs2, ss0, ss1, ss2, zsem):
    cid = lax.axis_index("c")
    sid = lax.axis_index("s")
    wid = sid * _NC + cid

    pltpu.async_copy(z_hbm, aggsh.at[pl.ds(sid * _RPT, _RPT)], zsem)

    pltpu.sync_copy(idx_hbm.at[wid], idxv)

    def _unpack(jj, sdb):
        for k in range(_C // 16):
            x = idxv[jj, pl.ds(k * 16, 16)]
            sdb[0, pl.ds(k * 16, 16)] = lax.bitwise_and(x, 0xFFFF)
            sdb[1, pl.ds(k * 16, 16)] = lax.shift_right_logical(x, 16)

    @pl.when(sid == 0)
    def _():
        pltpu.sync_copy(
            z_hbm.at[pl.ds(0, _REM)], aggsh.at[pl.ds(_NS * _RPT, _REM)]
        )

    slots = (
        (sdb0, rb0, gs0, ss0),
        (sdb1, rb1, gs1, ss1),
        (sdb2, rb2, gs2, ss2),
    )

    _unpack(0, sdb0)
    pltpu.async_copy(m_hbm.at[sdb0.at[0]], rb0, gs0)
    _unpack(1, sdb1)
    pltpu.async_copy(m_hbm.at[sdb1.at[0]], rb1, gs1)

    pltpu.make_async_copy(z_hbm, aggsh.at[pl.ds(sid * _RPT, _RPT)], zsem).wait()
    plsc.subcore_barrier()

    def _body(k, r):
        sdb, rb, gs, ss = slots[r]
        sdb2, rb2, gs2, ss2_ = slots[(r + 2) % 3]
        pltpu.make_async_copy(m_hbm.at[sdb.at[0]], rb, gs).wait()
        pltpu.async_copy(rb, aggsh.at[sdb.at[1]], ss, add=True)

        @pl.when(k >= 1)
        def _():
            pltpu.make_async_copy(rb2, aggsh.at[sdb2.at[1]], ss2_).wait()

        @pl.when(k + 2 < _NCH)
        def _():
            _unpack(k + 2, sdb2)
            pltpu.async_copy(m_hbm.at[sdb2.at[0]], rb2, gs2)

    def _tri(i3, carry):
        k0 = i3 * 3
        for r in range(3):
            _body(k0 + r, r)
        return carry

    _ntri = (_NCH - 2) // 3
    assert _ntri * 3 + 2 == _NCH
    lax.fori_loop(0, _ntri, _tri, 0)
    _body(_NCH - 2, 0)
    _body(_NCH - 1, 1)
    pltpu.make_async_copy(rb1, aggsh.at[sdb1.at[1]], ss1).wait()

    plsc.subcore_barrier()
    pltpu.sync_copy(
        aggsh.at[pl.ds(sid * _RPT, _RPT)],
        out_hbm.at[cid, pl.ds(sid * _RPT, _RPT)],
    )

    @pl.when(sid == 0)
    def _():
        pltpu.sync_copy(
            aggsh.at[pl.ds(_NS * _RPT, _REM)],
            out_hbm.at[cid, pl.ds(_NS * _RPT, _REM)],
        )


def _gru_msg_body(a_ref, h_ref, wih_ref, whh_ref, bih_ref, bhh_ref,
                  wm_ref, bm_ref, o_ref, m_ref):
    agg = a_ref[0] + a_ref[1]
    h = h_ref[...]
    gi = (
        jnp.dot(agg, wih_ref[...], preferred_element_type=jnp.float32)
        + bih_ref[...]
    )
    gh = (
        jnp.dot(h, whh_ref[...], preferred_element_type=jnp.float32)
        + bhh_ref[...]
    )
    rg = jax.nn.sigmoid(gi[:, :_DIM] + gh[:, :_DIM])
    zg = jax.nn.sigmoid(gi[:, _DIM:2 * _DIM] + gh[:, _DIM:2 * _DIM])
    ng = jnp.tanh(gi[:, 2 * _DIM:] + rg * gh[:, 2 * _DIM:])
    h_new = (1.0 - zg) * ng + zg * h
    o_ref[...] = h_new
    for e in range(_NET):
        m_ref[e] = (
            jnp.dot(h_new, wm_ref[:, e * _DIM:(e + 1) * _DIM],
                    preferred_element_type=jnp.float32)
            + bm_ref[:, e * _DIM:(e + 1) * _DIM]
        )


def _gru_msg(agg2, h, wih_t, whh_t, bih2d, bhh2d, wm_t, bm2d):
    rb = 1000
    return pl.pallas_call(
        _gru_msg_body,
        grid=(_N // rb,),
        in_specs=[
            pl.BlockSpec((_NC, rb, _DIM), lambda i: (0, i, 0)),
            pl.BlockSpec((rb, _DIM), lambda i: (i, 0)),
            pl.BlockSpec((_DIM, 3 * _DIM), lambda i: (0, 0)),
            pl.BlockSpec((_DIM, 3 * _DIM), lambda i: (0, 0)),
            pl.BlockSpec((1, 3 * _DIM), lambda i: (0, 0)),
            pl.BlockSpec((1, 3 * _DIM), lambda i: (0, 0)),
            pl.BlockSpec((_DIM, _NET * _DIM), lambda i: (0, 0)),
            pl.BlockSpec((1, _NET * _DIM), lambda i: (0, 0)),
        ],
        out_specs=[
            pl.BlockSpec((rb, _DIM), lambda i: (i, 0)),
            pl.BlockSpec((_NET, rb, _DIM), lambda i: (0, i, 0)),
        ],
        out_shape=[
            jax.ShapeDtypeStruct((_N, _DIM), jnp.float32),
            jax.ShapeDtypeStruct((_NET, _N, _DIM), jnp.float32),
        ],
    )(agg2, h, wih_t, whh_t, bih2d, bhh2d, wm_t, bm2d)


def _gru_body(a_ref, h_ref, wih_ref, whh_ref, bih_ref, bhh_ref, o_ref):
    agg = a_ref[0] + a_ref[1]
    h = h_ref[...]
    gi = (
        jnp.dot(agg, wih_ref[...], preferred_element_type=jnp.float32)
        + bih_ref[...]
    )
    gh = (
        jnp.dot(h, whh_ref[...], preferred_element_type=jnp.float32)
        + bhh_ref[...]
    )
    rg = jax.nn.sigmoid(gi[:, :_DIM] + gh[:, :_DIM])
    zg = jax.nn.sigmoid(gi[:, _DIM:2 * _DIM] + gh[:, _DIM:2 * _DIM])
    ng = jnp.tanh(gi[:, 2 * _DIM:] + rg * gh[:, 2 * _DIM:])
    o_ref[...] = (1.0 - zg) * ng + zg * h


def _gru(agg2, h, wih_t, whh_t, bih2d, bhh2d):
    rb = 1000
    return pl.pallas_call(
        _gru_body,
        grid=(_N // rb,),
        in_specs=[
            pl.BlockSpec((_NC, rb, _DIM), lambda i: (0, i, 0)),
            pl.BlockSpec((rb, _DIM), lambda i: (i, 0)),
            pl.BlockSpec((_DIM, 3 * _DIM), lambda i: (0, 0)),
            pl.BlockSpec((_DIM, 3 * _DIM), lambda i: (0, 0)),
            pl.BlockSpec((1, 3 * _DIM), lambda i: (0, 0)),
            pl.BlockSpec((1, 3 * _DIM), lambda i: (0, 0)),
        ],
        out_specs=pl.BlockSpec((rb, _DIM), lambda i: (i, 0)),
        out_shape=jax.ShapeDtypeStruct((_N, _DIM), jnp.float32),
    )(agg2, h, wih_t, whh_t, bih2d, bhh2d)


def kernel(embedding, edges, W_msg, b_msg, W_ih, W_hh, b_ih, b_hh):
    wm_t = W_msg.T
    wih_t = W_ih.T
    whh_t = W_hh.T
    bm2d = b_msg.reshape(1, _NET * _DIM)
    bih2d = b_ih.reshape(1, 3 * _DIM)
    bhh2d = b_hh.reshape(1, 3 * _DIM)

    e = edges.astype(jnp.int32)
    src = e[:, 2] * _N + e[:, 0]
    dst = e[:, 1]
    packed = jnp.bitwise_or(jnp.left_shift(dst, 16), src).reshape(
        _NW, _NCH, _C)
    zrows = jnp.zeros((_RPT, _DIM), jnp.float32)

    h = embedding
    m = _msg(h, wm_t, bm2d)
    for step in range(_NSTEPS):
        agg2 = _sc_agg(m.reshape(_N * _NET, _DIM), packed, zrows)
        if step < _NSTEPS - 1:
            h, m = _gru_msg(agg2, h, wih_t, whh_t, bih2d, bhh2d, wm_t, bm2d)
        else:
            h = _gru(agg2, h, wih_t, whh_t, bih2d, bhh2d)
    return h


# speedup vs baseline: 1.6669x; 1.0299x over previous
import functools

import jax
import jax.numpy as jnp
from jax import lax
from jax.experimental import pallas as pl
from jax.experimental.pallas import tpu as pltpu
from jax.experimental.pallas import tpu_sc as plsc

_N = 10000
_E = 320000
_DIM = 128
_NET = 4
_NSTEPS = 4

_NC = 2
_NS = 16
_NW = _NC * _NS
_EPW = _E // _NW
_C = 80
_NCH = _EPW // _C
_RPT = 624
_REM = _N - _NS * _RPT


def _msg_body(h_ref, w_ref, b_ref, o_ref):
    h = h_ref[...]
    for e in range(_NET):
        o_ref[e] = (
            jnp.dot(h, w_ref[:, e * _DIM:(e + 1) * _DIM],
                    preferred_element_type=jnp.float32)
            + b_ref[:, e * _DIM:(e + 1) * _DIM]
        )


def _msg(h, w_t, b2d):
    rb = 1000
    return pl.pallas_call(
        _msg_body,
        grid=(_N // rb,),
        in_specs=[
            pl.BlockSpec((rb, _DIM), lambda i: (i, 0)),
            pl.BlockSpec((_DIM, _NET * _DIM), lambda i: (0, 0)),
            pl.BlockSpec((1, _NET * _DIM), lambda i: (0, 0)),
        ],
        out_specs=pl.BlockSpec((_NET, rb, _DIM), lambda i: (0, i, 0)),
        out_shape=jax.ShapeDtypeStruct((_NET, _N, _DIM), jnp.float32),
    )(h, w_t, b2d)


@functools.partial(
    pl.kernel,
    out_type=jax.ShapeDtypeStruct((_NC, _N, _DIM), jnp.float32),
    mesh=plsc.VectorSubcoreMesh(core_axis_name="c", subcore_axis_name="s"),
    scratch_types=[
        pltpu.VMEM((_NCH, _C), jnp.int32),
        pltpu.VMEM((2, _C), jnp.int32),
        pltpu.VMEM((2, _C), jnp.int32),
        pltpu.VMEM((2, _C), jnp.int32),
        pltpu.VMEM((_C, _DIM), jnp.float32),
        pltpu.VMEM((_C, _DIM), jnp.float32),
        pltpu.VMEM((_C, _DIM), jnp.float32),
        pltpu.VMEM_SHARED((_N, _DIM), jnp.float32),
    ] + [pltpu.SemaphoreType.DMA] * 6,
)
def _sc_agg(m_hbm, idx_hbm, out_hbm,
            idxv, sdb0, sdb1, sdb2, rb0, rb1, rb2, aggsh,
            gs0, gs1, gs2, ss0, ss1, ss2):
    cid = lax.axis_index("c")
    sid = lax.axis_index("s")
    wid = sid * _NC + cid

    pltpu.sync_copy(idx_hbm.at[wid], idxv)

    def _unpack(jj, sdb):
        for k in range(_C // 16):
            x = idxv[jj, pl.ds(k * 16, 16)]
            sdb[0, pl.ds(k * 16, 16)] = lax.bitwise_and(x, 0xFFFF)
            sdb[1, pl.ds(k * 16, 16)] = lax.shift_right_logical(x, 16)

    slots = (
        (sdb0, rb0, gs0, ss0),
        (sdb1, rb1, gs1, ss1),
        (sdb2, rb2, gs2, ss2),
    )

    _unpack(0, sdb0)
    pltpu.async_copy(m_hbm.at[sdb0.at[0]], rb0, gs0)
    _unpack(1, sdb1)
    pltpu.async_copy(m_hbm.at[sdb1.at[0]], rb1, gs1)

    zv = jnp.zeros((16,), jnp.float32)

    def _zrow(r, carry):
        for j in range(_DIM // 16):
            rb2[r, pl.ds(j * 16, 16)] = zv
        return carry

    lax.fori_loop(0, _C, _zrow, 0)
    base = sid * _RPT
    for i in range(_RPT // _C):
        pltpu.sync_copy(rb2, aggsh.at[pl.ds(base + i * _C, _C)])
    _TAIL = _RPT % _C
    pltpu.sync_copy(
        rb2.at[pl.ds(0, _TAIL)],
        aggsh.at[pl.ds(base + (_RPT // _C) * _C, _TAIL)],
    )

    @pl.when(sid == 0)
    def _():
        pltpu.sync_copy(
            rb2.at[pl.ds(0, _REM)], aggsh.at[pl.ds(_NS * _RPT, _REM)]
        )

    plsc.subcore_barrier()

    def _body(k, r):
        sdb, rb, gs, ss = slots[r]
        sdb2, rb2, gs2, ss2_ = slots[(r + 2) % 3]
        pltpu.make_async_copy(m_hbm.at[sdb.at[0]], rb, gs).wait()
        pltpu.async_copy(rb, aggsh.at[sdb.at[1]], ss, add=True)

        @pl.when(k >= 1)
        def _():
            pltpu.make_async_copy(rb2, aggsh.at[sdb2.at[1]], ss2_).wait()

        @pl.when(k + 2 < _NCH)
        def _():
            _unpack(k + 2, sdb2)
            pltpu.async_copy(m_hbm.at[sdb2.at[0]], rb2, gs2)

    def _tri(i3, carry):
        k0 = i3 * 3
        for r in range(3):
            _body(k0 + r, r)
        return carry

    _ntri = (_NCH - 2) // 3
    assert _ntri * 3 + 2 == _NCH
    lax.fori_loop(0, _ntri, _tri, 0)
    _body(_NCH - 2, 0)
    _body(_NCH - 1, 1)
    pltpu.make_async_copy(rb1, aggsh.at[sdb1.at[1]], ss1).wait()

    plsc.subcore_barrier()
    pltpu.sync_copy(
        aggsh.at[pl.ds(sid * _RPT, _RPT)],
        out_hbm.at[cid, pl.ds(sid * _RPT, _RPT)],
    )

    @pl.when(sid == 0)
    def _():
        pltpu.sync_copy(
            aggsh.at[pl.ds(_NS * _RPT, _REM)],
            out_hbm.at[cid, pl.ds(_NS * _RPT, _REM)],
        )


def _gru_msg_body(a_ref, h_ref, wih_ref, whh_ref, bih_ref, bhh_ref,
                  wm_ref, bm_ref, o_ref, m_ref):
    agg = a_ref[0] + a_ref[1]
    h = h_ref[...]
    gi = (
        jnp.dot(agg, wih_ref[...], preferred_element_type=jnp.float32)
        + bih_ref[...]
    )
    gh = (
        jnp.dot(h, whh_ref[...], preferred_element_type=jnp.float32)
        + bhh_ref[...]
    )
    rg = jax.nn.sigmoid(gi[:, :_DIM] + gh[:, :_DIM])
    zg = jax.nn.sigmoid(gi[:, _DIM:2 * _DIM] + gh[:, _DIM:2 * _DIM])
    ng = jnp.tanh(gi[:, 2 * _DIM:] + rg * gh[:, 2 * _DIM:])
    h_new = (1.0 - zg) * ng + zg * h
    o_ref[...] = h_new
    for e in range(_NET):
        m_ref[e] = (
            jnp.dot(h_new, wm_ref[:, e * _DIM:(e + 1) * _DIM],
                    preferred_element_type=jnp.float32)
            + bm_ref[:, e * _DIM:(e + 1) * _DIM]
        )


def _gru_msg(agg2, h, wih_t, whh_t, bih2d, bhh2d, wm_t, bm2d):
    rb = 1000
    return pl.pallas_call(
        _gru_msg_body,
        grid=(_N // rb,),
        in_specs=[
            pl.BlockSpec((_NC, rb, _DIM), lambda i: (0, i, 0)),
            pl.BlockSpec((rb, _DIM), lambda i: (i, 0)),
            pl.BlockSpec((_DIM, 3 * _DIM), lambda i: (0, 0)),
            pl.BlockSpec((_DIM, 3 * _DIM), lambda i: (0, 0)),
            pl.BlockSpec((1, 3 * _DIM), lambda i: (0, 0)),
            pl.BlockSpec((1, 3 * _DIM), lambda i: (0, 0)),
            pl.BlockSpec((_DIM, _NET * _DIM), lambda i: (0, 0)),
            pl.BlockSpec((1, _NET * _DIM), lambda i: (0, 0)),
        ],
        out_specs=[
            pl.BlockSpec((rb, _DIM), lambda i: (i, 0)),
            pl.BlockSpec((_NET, rb, _DIM), lambda i: (0, i, 0)),
        ],
        out_shape=[
            jax.ShapeDtypeStruct((_N, _DIM), jnp.float32),
            jax.ShapeDtypeStruct((_NET, _N, _DIM), jnp.float32),
        ],
    )(agg2, h, wih_t, whh_t, bih2d, bhh2d, wm_t, bm2d)


def _gru_body(a_ref, h_ref, wih_ref, whh_ref, bih_ref, bhh_ref, o_ref):
    agg = a_ref[0] + a_ref[1]
    h = h_ref[...]
    gi = (
        jnp.dot(agg, wih_ref[...], preferred_element_type=jnp.float32)
        + bih_ref[...]
    )
    gh = (
        jnp.dot(h, whh_ref[...], preferred_element_type=jnp.float32)
        + bhh_ref[...]
    )
    rg = jax.nn.sigmoid(gi[:, :_DIM] + gh[:, :_DIM])
    zg = jax.nn.sigmoid(gi[:, _DIM:2 * _DIM] + gh[:, _DIM:2 * _DIM])
    ng = jnp.tanh(gi[:, 2 * _DIM:] + rg * gh[:, 2 * _DIM:])
    o_ref[...] = (1.0 - zg) * ng + zg * h


def _gru(agg2, h, wih_t, whh_t, bih2d, bhh2d):
    rb = 1000
    return pl.pallas_call(
        _gru_body,
        grid=(_N // rb,),
        in_specs=[
            pl.BlockSpec((_NC, rb, _DIM), lambda i: (0, i, 0)),
            pl.BlockSpec((rb, _DIM), lambda i: (i, 0)),
            pl.BlockSpec((_DIM, 3 * _DIM), lambda i: (0, 0)),
            pl.BlockSpec((_DIM, 3 * _DIM), lambda i: (0, 0)),
            pl.BlockSpec((1, 3 * _DIM), lambda i: (0, 0)),
            pl.BlockSpec((1, 3 * _DIM), lambda i: (0, 0)),
        ],
        out_specs=pl.BlockSpec((rb, _DIM), lambda i: (i, 0)),
        out_shape=jax.ShapeDtypeStruct((_N, _DIM), jnp.float32),
    )(agg2, h, wih_t, whh_t, bih2d, bhh2d)


def kernel(embedding, edges, W_msg, b_msg, W_ih, W_hh, b_ih, b_hh):
    wm_t = W_msg.T
    wih_t = W_ih.T
    whh_t = W_hh.T
    bm2d = b_msg.reshape(1, _NET * _DIM)
    bih2d = b_ih.reshape(1, 3 * _DIM)
    bhh2d = b_hh.reshape(1, 3 * _DIM)

    e = edges.astype(jnp.int32)
    src = e[:, 2] * _N + e[:, 0]
    dst = e[:, 1]
    packed = jnp.bitwise_or(jnp.left_shift(dst, 16), src).reshape(
        _NW, _NCH, _C)

    h = embedding
    m = _msg(h, wm_t, bm2d)
    for step in range(_NSTEPS):
        agg2 = _sc_agg(m.reshape(_N * _NET, _DIM), packed)
        if step < _NSTEPS - 1:
            h, m = _gru_msg(agg2, h, wih_t, whh_t, bih2d, bhh2d, wm_t, bm2d)
        else:
            h = _gru(agg2, h, wih_t, whh_t, bih2d, bhh2d)
    return h


# speedup vs baseline: 1.6750x; 1.0048x over previous
import functools

import jax
import jax.numpy as jnp
from jax import lax
from jax.experimental import pallas as pl
from jax.experimental.pallas import tpu as pltpu
from jax.experimental.pallas import tpu_sc as plsc

_N = 10000
_E = 320000
_DIM = 128
_NET = 4
_NSTEPS = 4

_NC = 2
_NS = 16
_NW = _NC * _NS
_EPW = _E // _NW
_C = 80
_NCH = _EPW // _C
_RPT = 624
_REM = _N - _NS * _RPT


def _msg_body(h_ref, w_ref, b_ref, o_ref):
    h = h_ref[...]
    for e in range(_NET):
        o_ref[e] = (
            jnp.dot(h, w_ref[:, e * _DIM:(e + 1) * _DIM],
                    preferred_element_type=jnp.float32)
            + b_ref[:, e * _DIM:(e + 1) * _DIM]
        )


def _msg(h, w_t, b2d):
    rb = 1000
    return pl.pallas_call(
        _msg_body,
        grid=(_N // rb,),
        in_specs=[
            pl.BlockSpec((rb, _DIM), lambda i: (i, 0)),
            pl.BlockSpec((_DIM, _NET * _DIM), lambda i: (0, 0)),
            pl.BlockSpec((1, _NET * _DIM), lambda i: (0, 0)),
        ],
        out_specs=pl.BlockSpec((_NET, rb, _DIM), lambda i: (0, i, 0)),
        out_shape=jax.ShapeDtypeStruct((_NET, _N, _DIM), jnp.float32),
    )(h, w_t, b2d)


@functools.partial(
    pl.kernel,
    out_type=jax.ShapeDtypeStruct((_NC, _N, _DIM), jnp.float32),
    mesh=plsc.VectorSubcoreMesh(core_axis_name="c", subcore_axis_name="s"),
    scratch_types=[
        pltpu.VMEM((_EPW,), jnp.int32),
        pltpu.VMEM((2, _C), jnp.int32),
        pltpu.VMEM((2, _C), jnp.int32),
        pltpu.VMEM((2, _C), jnp.int32),
        pltpu.VMEM((_C, _DIM), jnp.float32),
        pltpu.VMEM((_C, _DIM), jnp.float32),
        pltpu.VMEM((_C, _DIM), jnp.float32),
        pltpu.VMEM_SHARED((_N, _DIM), jnp.float32),
    ] + [pltpu.SemaphoreType.DMA] * 6,
)
def _sc_agg(m_hbm, idx_hbm, out_hbm,
            idxv, sdb0, sdb1, sdb2, rb0, rb1, rb2, aggsh,
            gs0, gs1, gs2, ss0, ss1, ss2):
    cid = lax.axis_index("c")
    sid = lax.axis_index("s")
    wid = sid * _NC + cid

    pltpu.sync_copy(idx_hbm.at[pl.ds(wid * _EPW, _EPW)], idxv)

    def _unpack(jj, sdb):
        for k in range(_C // 16):
            x = idxv[pl.ds(jj * _C + k * 16, 16)]
            sdb[0, pl.ds(k * 16, 16)] = lax.bitwise_and(x, 0xFFFF)
            sdb[1, pl.ds(k * 16, 16)] = lax.shift_right_logical(x, 16)

    slots = (
        (sdb0, rb0, gs0, ss0),
        (sdb1, rb1, gs1, ss1),
        (sdb2, rb2, gs2, ss2),
    )

    _unpack(0, sdb0)
    pltpu.async_copy(m_hbm.at[sdb0.at[0]], rb0, gs0)
    _unpack(1, sdb1)
    pltpu.async_copy(m_hbm.at[sdb1.at[0]], rb1, gs1)

    zv = jnp.zeros((16,), jnp.float32)

    def _zrow(r, carry):
        for j in range(_DIM // 16):
            rb2[r, pl.ds(j * 16, 16)] = zv
        return carry

    lax.fori_loop(0, _C, _zrow, 0)
    base = sid * _RPT
    for i in range(_RPT // _C):
        pltpu.sync_copy(rb2, aggsh.at[pl.ds(base + i * _C, _C)])
    _TAIL = _RPT % _C
    pltpu.sync_copy(
        rb2.at[pl.ds(0, _TAIL)],
        aggsh.at[pl.ds(base + (_RPT // _C) * _C, _TAIL)],
    )

    @pl.when(sid == 0)
    def _():
        pltpu.sync_copy(
            rb2.at[pl.ds(0, _REM)], aggsh.at[pl.ds(_NS * _RPT, _REM)]
        )

    plsc.subcore_barrier()

    def _body(k, r):
        sdb, rb, gs, ss = slots[r]
        sdb2, rb2, gs2, ss2_ = slots[(r + 2) % 3]
        pltpu.make_async_copy(m_hbm.at[sdb.at[0]], rb, gs).wait()
        pltpu.async_copy(rb, aggsh.at[sdb.at[1]], ss, add=True)

        @pl.when(k >= 1)
        def _():
            pltpu.make_async_copy(rb2, aggsh.at[sdb2.at[1]], ss2_).wait()

        @pl.when(k + 2 < _NCH)
        def _():
            _unpack(k + 2, sdb2)
            pltpu.async_copy(m_hbm.at[sdb2.at[0]], rb2, gs2)

    def _tri(i3, carry):
        k0 = i3 * 3
        for r in range(3):
            _body(k0 + r, r)
        return carry

    _ntri = (_NCH - 2) // 3
    assert _ntri * 3 + 2 == _NCH
    lax.fori_loop(0, _ntri, _tri, 0)
    _body(_NCH - 2, 0)
    _body(_NCH - 1, 1)
    pltpu.make_async_copy(rb1, aggsh.at[sdb1.at[1]], ss1).wait()

    plsc.subcore_barrier()
    pltpu.sync_copy(
        aggsh.at[pl.ds(sid * _RPT, _RPT)],
        out_hbm.at[cid, pl.ds(sid * _RPT, _RPT)],
    )

    @pl.when(sid == 0)
    def _():
        pltpu.sync_copy(
            aggsh.at[pl.ds(_NS * _RPT, _REM)],
            out_hbm.at[cid, pl.ds(_NS * _RPT, _REM)],
        )


def _gru_msg_body(a_ref, h_ref, wih_ref, whh_ref, bih_ref, bhh_ref,
                  wm_ref, bm_ref, o_ref, m_ref):
    agg = a_ref[0] + a_ref[1]
    h = h_ref[...]
    gi = (
        jnp.dot(agg, wih_ref[...], preferred_element_type=jnp.float32)
        + bih_ref[...]
    )
    gh = (
        jnp.dot(h, whh_ref[...], preferred_element_type=jnp.float32)
        + bhh_ref[...]
    )
    rg = jax.nn.sigmoid(gi[:, :_DIM] + gh[:, :_DIM])
    zg = jax.nn.sigmoid(gi[:, _DIM:2 * _DIM] + gh[:, _DIM:2 * _DIM])
    ng = jnp.tanh(gi[:, 2 * _DIM:] + rg * gh[:, 2 * _DIM:])
    h_new = (1.0 - zg) * ng + zg * h
    o_ref[...] = h_new
    for e in range(_NET):
        m_ref[e] = (
            jnp.dot(h_new, wm_ref[:, e * _DIM:(e + 1) * _DIM],
                    preferred_element_type=jnp.float32)
            + bm_ref[:, e * _DIM:(e + 1) * _DIM]
        )


def _gru_msg(agg2, h, wih_t, whh_t, bih2d, bhh2d, wm_t, bm2d):
    rb = 1000
    return pl.pallas_call(
        _gru_msg_body,
        grid=(_N // rb,),
        in_specs=[
            pl.BlockSpec((_NC, rb, _DIM), lambda i: (0, i, 0)),
            pl.BlockSpec((rb, _DIM), lambda i: (i, 0)),
            pl.BlockSpec((_DIM, 3 * _DIM), lambda i: (0, 0)),
            pl.BlockSpec((_DIM, 3 * _DIM), lambda i: (0, 0)),
            pl.BlockSpec((1, 3 * _DIM), lambda i: (0, 0)),
            pl.BlockSpec((1, 3 * _DIM), lambda i: (0, 0)),
            pl.BlockSpec((_DIM, _NET * _DIM), lambda i: (0, 0)),
            pl.BlockSpec((1, _NET * _DIM), lambda i: (0, 0)),
        ],
        out_specs=[
            pl.BlockSpec((rb, _DIM), lambda i: (i, 0)),
            pl.BlockSpec((_NET, rb, _DIM), lambda i: (0, i, 0)),
        ],
        out_shape=[
            jax.ShapeDtypeStruct((_N, _DIM), jnp.float32),
            jax.ShapeDtypeStruct((_NET, _N, _DIM), jnp.float32),
        ],
    )(agg2, h, wih_t, whh_t, bih2d, bhh2d, wm_t, bm2d)


def _gru_body(a_ref, h_ref, wih_ref, whh_ref, bih_ref, bhh_ref, o_ref):
    agg = a_ref[0] + a_ref[1]
    h = h_ref[...]
    gi = (
        jnp.dot(agg, wih_ref[...], preferred_element_type=jnp.float32)
        + bih_ref[...]
    )
    gh = (
        jnp.dot(h, whh_ref[...], preferred_element_type=jnp.float32)
        + bhh_ref[...]
    )
    rg = jax.nn.sigmoid(gi[:, :_DIM] + gh[:, :_DIM])
    zg = jax.nn.sigmoid(gi[:, _DIM:2 * _DIM] + gh[:, _DIM:2 * _DIM])
    ng = jnp.tanh(gi[:, 2 * _DIM:] + rg * gh[:, 2 * _DIM:])
    o_ref[...] = (1.0 - zg) * ng + zg * h


def _gru(agg2, h, wih_t, whh_t, bih2d, bhh2d):
    rb = 1000
    return pl.pallas_call(
        _gru_body,
        grid=(_N // rb,),
        in_specs=[
            pl.BlockSpec((_NC, rb, _DIM), lambda i: (0, i, 0)),
            pl.BlockSpec((rb, _DIM), lambda i: (i, 0)),
            pl.BlockSpec((_DIM, 3 * _DIM), lambda i: (0, 0)),
            pl.BlockSpec((_DIM, 3 * _DIM), lambda i: (0, 0)),
            pl.BlockSpec((1, 3 * _DIM), lambda i: (0, 0)),
            pl.BlockSpec((1, 3 * _DIM), lambda i: (0, 0)),
        ],
        out_specs=pl.BlockSpec((rb, _DIM), lambda i: (i, 0)),
        out_shape=jax.ShapeDtypeStruct((_N, _DIM), jnp.float32),
    )(agg2, h, wih_t, whh_t, bih2d, bhh2d)


def kernel(embedding, edges, W_msg, b_msg, W_ih, W_hh, b_ih, b_hh):
    wm_t = W_msg.T
    wih_t = W_ih.T
    whh_t = W_hh.T
    bm2d = b_msg.reshape(1, _NET * _DIM)
    bih2d = b_ih.reshape(1, 3 * _DIM)
    bhh2d = b_hh.reshape(1, 3 * _DIM)

    e = edges.astype(jnp.int32)
    src = e[:, 2] * _N + e[:, 0]
    dst = e[:, 1]
    packed = jnp.bitwise_or(jnp.left_shift(dst, 16), src)

    h = embedding
    m = _msg(h, wm_t, bm2d)
    for step in range(_NSTEPS):
        agg2 = _sc_agg(m.reshape(_N * _NET, _DIM), packed)
        if step < _NSTEPS - 1:
            h, m = _gru_msg(agg2, h, wih_t, whh_t, bih2d, bhh2d, wm_t, bm2d)
        else:
            h = _gru(agg2, h, wih_t, whh_t, bih2d, bhh2d)
    return h
